# baseline (speedup 1.0000x reference)
import math

import jax
import jax.numpy as jnp
from jax import lax
from jax.experimental import pallas as pl
from jax.experimental.pallas import tpu as pltpu

T, B, I, H = 128, 128, 512, 512
KPAD = 8


def _bilstm_kernel(x_ref, wih_ref, whh_ref, b_ref, out_ref, h, c):
    t = pl.program_id(1)

    @pl.when(t == 0)
    def _init():
        h[...] = jnp.zeros_like(h)
        c[...] = jnp.zeros_like(c)

    g = (jnp.dot(x_ref[0], wih_ref[0], preferred_element_type=jnp.float32)
         + jnp.dot(h[...], whh_ref[0], preferred_element_type=jnp.float32)
         + b_ref[0])
    ig = jax.nn.sigmoid(g[:, 0:H])
    fg = jax.nn.sigmoid(g[:, H:2 * H])
    gg = jnp.tanh(g[:, 2 * H:3 * H])
    og = jax.nn.sigmoid(g[:, 3 * H:4 * H])
    c_new = fg * c[...] + ig * gg
    h_new = og * jnp.tanh(c_new)
    c[...] = c_new
    h[...] = h_new
    out_ref[0, 0] = h_new


def _align_topk_kernel(f_ref, b_ref, lstm_ref, vals_ref, idx_ref):
    lf = f_ref[0]
    lb = b_ref[0]
    lstm_ref[0, :, 0:H] = lf
    lstm_ref[0, :, H:2 * H] = lb
    dn = (((1,), (1,)), ((), ()))
    a = (lax.dot_general(lf, lf, dn, preferred_element_type=jnp.float32)
         + lax.dot_general(lb, lb, dn, preferred_element_type=jnp.float32))
    a = a * (1.0 / math.sqrt(2 * H))
    iota = lax.broadcasted_iota(jnp.int32, (T, T), 1)
    neg = jnp.float32(-3e38)
    vals, idxs = [], []
    for _ in range(3):
        m = jnp.max(a, axis=1, keepdims=True)
        sel = jnp.where(a == m, iota, T)
        ix = jnp.min(sel, axis=1, keepdims=True)
        vals.append(m)
        idxs.append(ix)
        a = jnp.where(iota == ix, neg, a)
    col = lax.broadcasted_iota(jnp.int32, (T, KPAD), 1)
    v = jnp.where(col == 0, vals[0],
                  jnp.where(col == 1, vals[1],
                            jnp.where(col == 2, vals[2], 0.0)))
    ii = jnp.where(col == 0, idxs[0],
                   jnp.where(col == 1, idxs[1],
                             jnp.where(col == 2, idxs[2], 0)))
    vals_ref[0] = v
    idx_ref[0] = ii


def kernel(sentences, W_ih_f, W_hh_f, b_ih_f, b_hh_f,
           W_ih_b, W_hh_b, b_ih_b, b_hh_b):
    wih = jnp.stack([W_ih_f.T, W_ih_b.T])
    whh = jnp.stack([W_hh_f.T, W_hh_b.T])
    bias = jnp.stack([(b_ih_f + b_hh_f).reshape(1, 4 * H),
                      (b_ih_b + b_hh_b).reshape(1, 4 * H)])

    def _t_eff(d, t):
        return jnp.where(d == 0, t, T - 1 - t)

    hs = pl.pallas_call(
        _bilstm_kernel,
        grid=(2, T),
        in_specs=[
            pl.BlockSpec((1, B, I), lambda d, t: (_t_eff(d, t), 0, 0)),
            pl.BlockSpec((1, I, 4 * H), lambda d, t: (d, 0, 0)),
            pl.BlockSpec((1, H, 4 * H), lambda d, t: (d, 0, 0)),
            pl.BlockSpec((1, 1, 4 * H), lambda d, t: (d, 0, 0)),
        ],
        out_specs=pl.BlockSpec((1, 1, B, H), lambda d, t: (d, _t_eff(d, t), 0, 0)),
        out_shape=jax.ShapeDtypeStruct((2, T, B, H), jnp.float32),
        scratch_shapes=[pltpu.VMEM((B, H), jnp.float32)] * 2,
        compiler_params=pltpu.CompilerParams(
            dimension_semantics=("parallel", "arbitrary"),
        ),
    )(sentences, wih, whh, bias)

    out_f = jnp.transpose(hs[0], (1, 0, 2))
    out_b = jnp.transpose(hs[1], (1, 0, 2))

    lstm_out, vals, idx = pl.pallas_call(
        _align_topk_kernel,
        grid=(B,),
        in_specs=[
            pl.BlockSpec((1, T, H), lambda b: (b, 0, 0)),
            pl.BlockSpec((1, T, H), lambda b: (b, 0, 0)),
        ],
        out_specs=[
            pl.BlockSpec((1, T, 2 * H), lambda b: (b, 0, 0)),
            pl.BlockSpec((1, T, KPAD), lambda b: (b, 0, 0)),
            pl.BlockSpec((1, T, KPAD), lambda b: (b, 0, 0)),
        ],
        out_shape=[
            jax.ShapeDtypeStruct((B, T, 2 * H), jnp.float32),
            jax.ShapeDtypeStruct((B, T, KPAD), jnp.float32),
            jax.ShapeDtypeStruct((B, T, KPAD), jnp.int32),
        ],
        compiler_params=pltpu.CompilerParams(
            dimension_semantics=("parallel",),
        ),
    )(out_f, out_b)

    adj = idx[:, :, :3].reshape(B, T * 3)
    row1 = jnp.broadcast_to(
        jnp.repeat(jnp.arange(T, dtype=jnp.int32), 3)[None, :], (B, T * 3))
    coo = jnp.stack([adj, row1], axis=1)
    return (coo, vals[:, :, :3], lstm_out)


# speedup vs baseline: 1.3768x; 1.3768x over previous
import math

import jax
import jax.numpy as jnp
from jax import lax
from jax.experimental import pallas as pl
from jax.experimental.pallas import tpu as pltpu

T, B, I, H = 128, 128, 512, 512
KPAD = 8


def _bilstm_kernel(xf_ref, xb_ref, wih_f_ref, whh_f_ref, bf_ref,
                   wih_b_ref, whh_b_ref, bb_ref,
                   outf_ref, outb_ref, hf, cf, hb, cb, accf, accb):
    t = pl.program_id(0)
    j = lax.rem(t, 8)

    @pl.when(t == 0)
    def _init():
        hf[...] = jnp.zeros_like(hf)
        cf[...] = jnp.zeros_like(cf)
        hb[...] = jnp.zeros_like(hb)
        cb[...] = jnp.zeros_like(cb)

    def _step(x, wih_ref, whh_ref, b_ref, h, c, acc, slot):
        g = (jnp.dot(x, wih_ref[...], preferred_element_type=jnp.float32)
             + jnp.dot(h[...], whh_ref[...], preferred_element_type=jnp.float32)
             + b_ref[...])
        ig = jax.nn.sigmoid(g[:, 0:H])
        fg = jax.nn.sigmoid(g[:, H:2 * H])
        gg = jnp.tanh(g[:, 2 * H:3 * H])
        og = jax.nn.sigmoid(g[:, 3 * H:4 * H])
        c_new = fg * c[...] + ig * gg
        h_new = og * jnp.tanh(c_new)
        c[...] = c_new
        h[...] = h_new
        acc[pl.ds(slot, 1)] = h_new[None]

    _step(xf_ref[0], wih_f_ref, whh_f_ref, bf_ref, hf, cf, accf, j)
    _step(xb_ref[0], wih_b_ref, whh_b_ref, bb_ref, hb, cb, accb, 7 - j)

    @pl.when(j == 7)
    def _flush():
        outf_ref[...] = jnp.transpose(accf[...], (1, 0, 2))
        outb_ref[...] = jnp.transpose(accb[...], (1, 0, 2))


def _align_topk_kernel(f_ref, b_ref, lstm_ref, vals_ref, idx_ref):
    lf = f_ref[0]
    lb = b_ref[0]
    lstm_ref[0, :, 0:H] = lf
    lstm_ref[0, :, H:2 * H] = lb
    dn = (((1,), (1,)), ((), ()))
    a = (lax.dot_general(lf, lf, dn, preferred_element_type=jnp.float32)
         + lax.dot_general(lb, lb, dn, preferred_element_type=jnp.float32))
    a = a * (1.0 / math.sqrt(2 * H))
    iota = lax.broadcasted_iota(jnp.int32, (T, T), 1)
    neg = jnp.float32(-3e38)
    vals, idxs = [], []
    for _ in range(3):
        m = jnp.max(a, axis=1, keepdims=True)
        sel = jnp.where(a == m, iota, T)
        ix = jnp.min(sel, axis=1, keepdims=True)
        vals.append(m)
        idxs.append(ix)
        a = jnp.where(iota == ix, neg, a)
    col = lax.broadcasted_iota(jnp.int32, (T, KPAD), 1)
    v = jnp.where(col == 0, vals[0],
                  jnp.where(col == 1, vals[1],
                            jnp.where(col == 2, vals[2], 0.0)))
    ii = jnp.where(col == 0, idxs[0],
                   jnp.where(col == 1, idxs[1],
                             jnp.where(col == 2, idxs[2], 0)))
    vals_ref[0] = v
    idx_ref[0] = ii


def kernel(sentences, W_ih_f, W_hh_f, b_ih_f, b_hh_f,
           W_ih_b, W_hh_b, b_ih_b, b_hh_b):
    wih_f = W_ih_f.T
    whh_f = W_hh_f.T
    wih_b = W_ih_b.T
    whh_b = W_hh_b.T
    bias_f = (b_ih_f + b_hh_f).reshape(1, 4 * H)
    bias_b = (b_ih_b + b_hh_b).reshape(1, 4 * H)

    out_f, out_b = pl.pallas_call(
        _bilstm_kernel,
        grid=(T,),
        in_specs=[
            pl.BlockSpec((1, B, I), lambda t: (t, 0, 0)),
            pl.BlockSpec((1, B, I), lambda t: (T - 1 - t, 0, 0)),
            pl.BlockSpec((I, 4 * H), lambda t: (0, 0)),
            pl.BlockSpec((H, 4 * H), lambda t: (0, 0)),
            pl.BlockSpec((1, 4 * H), lambda t: (0, 0)),
            pl.BlockSpec((I, 4 * H), lambda t: (0, 0)),
            pl.BlockSpec((H, 4 * H), lambda t: (0, 0)),
            pl.BlockSpec((1, 4 * H), lambda t: (0, 0)),
        ],
        out_specs=[
            pl.BlockSpec((B, 8, H), lambda t: (0, t // 8, 0)),
            pl.BlockSpec((B, 8, H), lambda t: (0, T // 8 - 1 - t // 8, 0)),
        ],
        out_shape=[
            jax.ShapeDtypeStruct((B, T, H), jnp.float32),
            jax.ShapeDtypeStruct((B, T, H), jnp.float32),
        ],
        scratch_shapes=([pltpu.VMEM((B, H), jnp.float32)] * 4
                        + [pltpu.VMEM((8, B, H), jnp.float32)] * 2),
        compiler_params=pltpu.CompilerParams(
            dimension_semantics=("arbitrary",),
        ),
    )(sentences, sentences, wih_f, whh_f, bias_f, wih_b, whh_b, bias_b)

    lstm_out, vals, idx = pl.pallas_call(
        _align_topk_kernel,
        grid=(B,),
        in_specs=[
            pl.BlockSpec((1, T, H), lambda b: (b, 0, 0)),
            pl.BlockSpec((1, T, H), lambda b: (b, 0, 0)),
        ],
        out_specs=[
            pl.BlockSpec((1, T, 2 * H), lambda b: (b, 0, 0)),
            pl.BlockSpec((1, T, KPAD), lambda b: (b, 0, 0)),
            pl.BlockSpec((1, T, KPAD), lambda b: (b, 0, 0)),
        ],
        out_shape=[
            jax.ShapeDtypeStruct((B, T, 2 * H), jnp.float32),
            jax.ShapeDtypeStruct((B, T, KPAD), jnp.float32),
            jax.ShapeDtypeStruct((B, T, KPAD), jnp.int32),
        ],
        compiler_params=pltpu.CompilerParams(
            dimension_semantics=("parallel",),
        ),
    )(out_f, out_b)

    adj = idx[:, :, :3].reshape(B, T * 3)
    row1 = jnp.broadcast_to(
        jnp.repeat(jnp.arange(T, dtype=jnp.int32), 3)[None, :], (B, T * 3))
    coo = jnp.stack([adj, row1], axis=1)
    return (coo, vals[:, :, :3], lstm_out)


# speedup vs baseline: 1.6012x; 1.1630x over previous
import math

import jax
import jax.numpy as jnp
from jax import lax
from jax.experimental import pallas as pl
from jax.experimental.pallas import tpu as pltpu

T, B, I, H = 128, 128, 512, 512
KPAD = 8


_CHUNK = 8


def _bilstm_kernel(xf_ref, xb_ref, wih_f_ref, whh_f_ref, bf_ref,
                   wih_b_ref, whh_b_ref, bb_ref,
                   outf_ref, outb_ref, hf, cf, hb, cb):
    k = pl.program_id(0)

    @pl.when(k == 0)
    def _init():
        hf[...] = jnp.zeros_like(hf)
        cf[...] = jnp.zeros_like(cf)
        hb[...] = jnp.zeros_like(hb)
        cb[...] = jnp.zeros_like(cb)

    xf = xf_ref[...].reshape(_CHUNK * B, I)
    xb = xb_ref[...].reshape(_CHUNK * B, I)
    gxf = jnp.dot(xf, wih_f_ref[...], preferred_element_type=jnp.float32)
    gxb = jnp.dot(xb, wih_b_ref[...], preferred_element_type=jnp.float32)

    def _cell(gx, whh_ref, b_ref, h, c):
        g = (gx + jnp.dot(h[...], whh_ref[...],
                          preferred_element_type=jnp.float32) + b_ref[...])
        ig = jax.nn.sigmoid(g[:, 0:H])
        fg = jax.nn.sigmoid(g[:, H:2 * H])
        gg = jnp.tanh(g[:, 2 * H:3 * H])
        og = jax.nn.sigmoid(g[:, 3 * H:4 * H])
        c_new = fg * c[...] + ig * gg
        h_new = og * jnp.tanh(c_new)
        c[...] = c_new
        h[...] = h_new
        return h_new

    hs_f, hs_b = [], []
    for i in range(_CHUNK):
        hs_f.append(_cell(gxf[i * B:(i + 1) * B], whh_f_ref, bf_ref, hf, cf))
        hs_b.append(_cell(gxb[(_CHUNK - 1 - i) * B:(_CHUNK - i) * B],
                          whh_b_ref, bb_ref, hb, cb))

    outf_ref[...] = jnp.stack(hs_f, axis=1)
    outb_ref[...] = jnp.stack(hs_b[::-1], axis=1)


_GB = 4


def _align_topk_kernel(f_ref, b_ref, lstm_ref, vals_ref, idx_ref):
    dn = (((1,), (1,)), ((), ()))
    iota = lax.broadcasted_iota(jnp.int32, (T, T), 1)
    col = lax.broadcasted_iota(jnp.int32, (T, KPAD), 1)
    neg = jnp.float32(-3e38)
    for g in range(_GB):
        lf = f_ref[g]
        lb = b_ref[g]
        lstm_ref[g, :, 0:H] = lf
        lstm_ref[g, :, H:2 * H] = lb
        a = (lax.dot_general(lf, lf, dn, preferred_element_type=jnp.float32)
             + lax.dot_general(lb, lb, dn, preferred_element_type=jnp.float32))
        a = a * (1.0 / math.sqrt(2 * H))
        vals, idxs = [], []
        for _ in range(3):
            m = jnp.max(a, axis=1, keepdims=True)
            sel = jnp.where(a == m, iota, T)
            ix = jnp.min(sel, axis=1, keepdims=True)
            vals.append(m)
            idxs.append(ix)
            a = jnp.where(iota == ix, neg, a)
        v = jnp.where(col == 0, vals[0],
                      jnp.where(col == 1, vals[1],
                                jnp.where(col == 2, vals[2], 0.0)))
        ii = jnp.where(col == 0, idxs[0],
                       jnp.where(col == 1, idxs[1],
                                 jnp.where(col == 2, idxs[2], 0)))
        vals_ref[g] = v
        idx_ref[g] = ii


def kernel(sentences, W_ih_f, W_hh_f, b_ih_f, b_hh_f,
           W_ih_b, W_hh_b, b_ih_b, b_hh_b):
    wih_f = W_ih_f.T
    whh_f = W_hh_f.T
    wih_b = W_ih_b.T
    whh_b = W_hh_b.T
    bias_f = (b_ih_f + b_hh_f).reshape(1, 4 * H)
    bias_b = (b_ih_b + b_hh_b).reshape(1, 4 * H)

    nk = T // _CHUNK
    out_f, out_b = pl.pallas_call(
        _bilstm_kernel,
        grid=(nk,),
        in_specs=[
            pl.BlockSpec((_CHUNK, B, I), lambda k: (k, 0, 0)),
            pl.BlockSpec((_CHUNK, B, I), lambda k: (nk - 1 - k, 0, 0)),
            pl.BlockSpec((I, 4 * H), lambda k: (0, 0)),
            pl.BlockSpec((H, 4 * H), lambda k: (0, 0)),
            pl.BlockSpec((1, 4 * H), lambda k: (0, 0)),
            pl.BlockSpec((I, 4 * H), lambda k: (0, 0)),
            pl.BlockSpec((H, 4 * H), lambda k: (0, 0)),
            pl.BlockSpec((1, 4 * H), lambda k: (0, 0)),
        ],
        out_specs=[
            pl.BlockSpec((B, _CHUNK, H), lambda k: (0, k, 0)),
            pl.BlockSpec((B, _CHUNK, H), lambda k: (0, nk - 1 - k, 0)),
        ],
        out_shape=[
            jax.ShapeDtypeStruct((B, T, H), jnp.float32),
            jax.ShapeDtypeStruct((B, T, H), jnp.float32),
        ],
        scratch_shapes=[pltpu.VMEM((B, H), jnp.float32)] * 4,
        compiler_params=pltpu.CompilerParams(
            dimension_semantics=("arbitrary",),
        ),
    )(sentences, sentences, wih_f, whh_f, bias_f, wih_b, whh_b, bias_b)

    lstm_out, vals, idx = pl.pallas_call(
        _align_topk_kernel,
        grid=(B // _GB,),
        in_specs=[
            pl.BlockSpec((_GB, T, H), lambda b: (b, 0, 0)),
            pl.BlockSpec((_GB, T, H), lambda b: (b, 0, 0)),
        ],
        out_specs=[
            pl.BlockSpec((_GB, T, 2 * H), lambda b: (b, 0, 0)),
            pl.BlockSpec((_GB, T, KPAD), lambda b: (b, 0, 0)),
            pl.BlockSpec((_GB, T, KPAD), lambda b: (b, 0, 0)),
        ],
        out_shape=[
            jax.ShapeDtypeStruct((B, T, 2 * H), jnp.float32),
            jax.ShapeDtypeStruct((B, T, KPAD), jnp.float32),
            jax.ShapeDtypeStruct((B, T, KPAD), jnp.int32),
        ],
        compiler_params=pltpu.CompilerParams(
            dimension_semantics=("parallel",),
        ),
    )(out_f, out_b)

    adj = idx[:, :, :3].reshape(B, T * 3)
    row1 = jnp.broadcast_to(
        jnp.repeat(jnp.arange(T, dtype=jnp.int32), 3)[None, :], (B, T * 3))
    coo = jnp.stack([adj, row1], axis=1)
    return (coo, vals[:, :, :3], lstm_out)


# speedup vs baseline: 1.6037x; 1.0016x over previous
import math

import jax
import jax.numpy as jnp
from jax import lax
from jax.experimental import pallas as pl
from jax.experimental.pallas import tpu as pltpu

T, B, I, H = 128, 128, 512, 512
KPAD = 8


_CHUNK = 8


def _bilstm_kernel(xf_ref, xb_ref, wih_f_ref, whh_f_ref, bf_ref,
                   wih_b_ref, whh_b_ref, bb_ref,
                   outf_ref, outb_ref, hf, cf, hb, cb):
    k = pl.program_id(0)

    @pl.when(k == 0)
    def _init():
        hf[...] = jnp.zeros_like(hf)
        cf[...] = jnp.zeros_like(cf)
        hb[...] = jnp.zeros_like(hb)
        cb[...] = jnp.zeros_like(cb)

    xf = xf_ref[...].reshape(_CHUNK * B, I)
    xb = xb_ref[...].reshape(_CHUNK * B, I)

    def _cell(gx, whh_ref, b_ref, h, c):
        g = (gx + jnp.dot(h[...], whh_ref[...],
                          preferred_element_type=jnp.float32) + b_ref[...])
        ig = jax.nn.sigmoid(g[:, 0:H])
        fg = jax.nn.sigmoid(g[:, H:2 * H])
        gg = jnp.tanh(g[:, 2 * H:3 * H])
        og = jax.nn.sigmoid(g[:, 3 * H:4 * H])
        c_new = fg * c[...] + ig * gg
        h_new = og * jnp.tanh(c_new)
        c[...] = c_new
        h[...] = h_new
        return h_new

    half = _CHUNK // 2
    hs_f, hs_b = [], []
    for p in range(2):
        f0 = p * half
        b_hi = _CHUNK - p * half
        b_lo = b_hi - half
        gxf = jnp.dot(xf[f0 * B:(f0 + half) * B], wih_f_ref[...],
                      preferred_element_type=jnp.float32)
        gxb = jnp.dot(xb[b_lo * B:b_hi * B], wih_b_ref[...],
                      preferred_element_type=jnp.float32)
        for i in range(half):
            hs_f.append(_cell(gxf[i * B:(i + 1) * B], whh_f_ref, bf_ref,
                              hf, cf))
            hs_b.append(_cell(gxb[(half - 1 - i) * B:(half - i) * B],
                              whh_b_ref, bb_ref, hb, cb))

    outf_ref[...] = jnp.stack(hs_f, axis=1)
    outb_ref[...] = jnp.stack(hs_b[::-1], axis=1)


_GB = 4


def _align_topk_kernel(f_ref, b_ref, lstm_ref, vals_ref, idx_ref):
    dn = (((1,), (1,)), ((), ()))
    iota = lax.broadcasted_iota(jnp.int32, (T, T), 1)
    col = lax.broadcasted_iota(jnp.int32, (T, KPAD), 1)
    neg = jnp.float32(-3e38)
    for g in range(_GB):
        lf = f_ref[g]
        lb = b_ref[g]
        lstm_ref[g, :, 0:H] = lf
        lstm_ref[g, :, H:2 * H] = lb
        a = (lax.dot_general(lf, lf, dn, preferred_element_type=jnp.float32)
             + lax.dot_general(lb, lb, dn, preferred_element_type=jnp.float32))
        a = a * (1.0 / math.sqrt(2 * H))
        vals, idxs = [], []
        for _ in range(3):
            m = jnp.max(a, axis=1, keepdims=True)
            sel = jnp.where(a == m, iota, T)
            ix = jnp.min(sel, axis=1, keepdims=True)
            vals.append(m)
            idxs.append(ix)
            a = jnp.where(iota == ix, neg, a)
        v = jnp.where(col == 0, vals[0],
                      jnp.where(col == 1, vals[1],
                                jnp.where(col == 2, vals[2], 0.0)))
        ii = jnp.where(col == 0, idxs[0],
                       jnp.where(col == 1, idxs[1],
                                 jnp.where(col == 2, idxs[2], 0)))
        vals_ref[g] = v
        idx_ref[g] = ii


def kernel(sentences, W_ih_f, W_hh_f, b_ih_f, b_hh_f,
           W_ih_b, W_hh_b, b_ih_b, b_hh_b):
    wih_f = W_ih_f.T
    whh_f = W_hh_f.T
    wih_b = W_ih_b.T
    whh_b = W_hh_b.T
    bias_f = (b_ih_f + b_hh_f).reshape(1, 4 * H)
    bias_b = (b_ih_b + b_hh_b).reshape(1, 4 * H)

    nk = T // _CHUNK
    out_f, out_b = pl.pallas_call(
        _bilstm_kernel,
        grid=(nk,),
        in_specs=[
            pl.BlockSpec((_CHUNK, B, I), lambda k: (k, 0, 0)),
            pl.BlockSpec((_CHUNK, B, I), lambda k: (nk - 1 - k, 0, 0)),
            pl.BlockSpec((I, 4 * H), lambda k: (0, 0)),
            pl.BlockSpec((H, 4 * H), lambda k: (0, 0)),
            pl.BlockSpec((1, 4 * H), lambda k: (0, 0)),
            pl.BlockSpec((I, 4 * H), lambda k: (0, 0)),
            pl.BlockSpec((H, 4 * H), lambda k: (0, 0)),
            pl.BlockSpec((1, 4 * H), lambda k: (0, 0)),
        ],
        out_specs=[
            pl.BlockSpec((B, _CHUNK, H), lambda k: (0, k, 0)),
            pl.BlockSpec((B, _CHUNK, H), lambda k: (0, nk - 1 - k, 0)),
        ],
        out_shape=[
            jax.ShapeDtypeStruct((B, T, H), jnp.float32),
            jax.ShapeDtypeStruct((B, T, H), jnp.float32),
        ],
        scratch_shapes=[pltpu.VMEM((B, H), jnp.float32)] * 4,
        compiler_params=pltpu.CompilerParams(
            dimension_semantics=("arbitrary",),
        ),
    )(sentences, sentences, wih_f, whh_f, bias_f, wih_b, whh_b, bias_b)

    lstm_out, vals, idx = pl.pallas_call(
        _align_topk_kernel,
        grid=(B // _GB,),
        in_specs=[
            pl.BlockSpec((_GB, T, H), lambda b: (b, 0, 0)),
            pl.BlockSpec((_GB, T, H), lambda b: (b, 0, 0)),
        ],
        out_specs=[
            pl.BlockSpec((_GB, T, 2 * H), lambda b: (b, 0, 0)),
            pl.BlockSpec((_GB, T, KPAD), lambda b: (b, 0, 0)),
            pl.BlockSpec((_GB, T, KPAD), lambda b: (b, 0, 0)),
        ],
        out_shape=[
            jax.ShapeDtypeStruct((B, T, 2 * H), jnp.float32),
            jax.ShapeDtypeStruct((B, T, KPAD), jnp.float32),
            jax.ShapeDtypeStruct((B, T, KPAD), jnp.int32),
        ],
        compiler_params=pltpu.CompilerParams(
            dimension_semantics=("parallel",),
        ),
    )(out_f, out_b)

    adj = idx[:, :, :3].reshape(B, T * 3)
    row1 = jnp.broadcast_to(
        jnp.repeat(jnp.arange(T, dtype=jnp.int32), 3)[None, :], (B, T * 3))
    coo = jnp.stack([adj, row1], axis=1)
    return (coo, vals[:, :, :3], lstm_out)


# speedup vs baseline: 1.6749x; 1.0444x over previous
import math

import jax
import jax.numpy as jnp
from jax import lax
from jax.experimental import pallas as pl
from jax.experimental.pallas import tpu as pltpu

T, B, I, H = 128, 128, 512, 512
KPAD = 8


_CHUNK = 8


def _bilstm_kernel(xf_ref, xb_ref, wih_f_ref, whh_f_ref, bf_ref,
                   wih_b_ref, whh_b_ref, bb_ref,
                   outf_ref, outb_ref, hf, cf, hb, cb):
    k = pl.program_id(0)

    @pl.when(k == 0)
    def _init():
        hf[...] = jnp.zeros_like(hf)
        cf[...] = jnp.zeros_like(cf)
        hb[...] = jnp.zeros_like(hb)
        cb[...] = jnp.zeros_like(cb)

    xf = xf_ref[...].reshape(_CHUNK * B, I)
    xb = xb_ref[...].reshape(_CHUNK * B, I)

    def _cell(x, wih_ref, whh_ref, b_ref, h, c):
        g = (jnp.dot(x, wih_ref[...], preferred_element_type=jnp.float32)
             + jnp.dot(h[...], whh_ref[...], preferred_element_type=jnp.float32)
             + b_ref[...])
        ig = jax.nn.sigmoid(g[:, 0:H])
        fg = jax.nn.sigmoid(g[:, H:2 * H])
        gg = jnp.tanh(g[:, 2 * H:3 * H])
        og = jax.nn.sigmoid(g[:, 3 * H:4 * H])
        c_new = fg * c[...] + ig * gg
        h_new = og * jnp.tanh(c_new)
        c[...] = c_new
        h[...] = h_new
        return h_new

    hs_f, hs_b = [], []
    for i in range(_CHUNK):
        hs_f.append(_cell(xf[i * B:(i + 1) * B],
                          wih_f_ref, whh_f_ref, bf_ref, hf, cf))
        hs_b.append(_cell(xb[(_CHUNK - 1 - i) * B:(_CHUNK - i) * B],
                          wih_b_ref, whh_b_ref, bb_ref, hb, cb))

    outf_ref[...] = jnp.stack(hs_f, axis=1)
    outb_ref[...] = jnp.stack(hs_b[::-1], axis=1)


_GB = 4


def _align_topk_kernel(f_ref, b_ref, lstm_ref, vals_ref, idx_ref):
    dn = (((1,), (1,)), ((), ()))
    iota = lax.broadcasted_iota(jnp.int32, (T, T), 1)
    neg = jnp.float32(-3e38)
    for g in range(_GB):
        lf = f_ref[g]
        lb = b_ref[g]
        lstm_ref[g, :, 0:H] = lf
        lstm_ref[g, :, H:2 * H] = lb
        a = (lax.dot_general(lf, lf, dn, preferred_element_type=jnp.float32)
             + lax.dot_general(lb, lb, dn, preferred_element_type=jnp.float32))
        a = a * (1.0 / math.sqrt(2 * H))
        vals_ref[g] = jnp.zeros((T, KPAD), jnp.float32)
        idx_ref[g] = jnp.zeros((T, KPAD), jnp.int32)
        for j in range(3):
            m = jnp.max(a, axis=1, keepdims=True)
            sel = jnp.where(a == m, iota, T)
            ix = jnp.min(sel, axis=1, keepdims=True)
            vals_ref[g, :, j:j + 1] = m
            idx_ref[g, :, j:j + 1] = ix
            if j < 2:
                a = jnp.where(iota == ix, neg, a)


def kernel(sentences, W_ih_f, W_hh_f, b_ih_f, b_hh_f,
           W_ih_b, W_hh_b, b_ih_b, b_hh_b):
    wih_f = W_ih_f.T
    whh_f = W_hh_f.T
    wih_b = W_ih_b.T
    whh_b = W_hh_b.T
    bias_f = (b_ih_f + b_hh_f).reshape(1, 4 * H)
    bias_b = (b_ih_b + b_hh_b).reshape(1, 4 * H)

    nk = T // _CHUNK
    out_f, out_b = pl.pallas_call(
        _bilstm_kernel,
        grid=(nk,),
        in_specs=[
            pl.BlockSpec((_CHUNK, B, I), lambda k: (k, 0, 0)),
            pl.BlockSpec((_CHUNK, B, I), lambda k: (nk - 1 - k, 0, 0)),
            pl.BlockSpec((I, 4 * H), lambda k: (0, 0)),
            pl.BlockSpec((H, 4 * H), lambda k: (0, 0)),
            pl.BlockSpec((1, 4 * H), lambda k: (0, 0)),
            pl.BlockSpec((I, 4 * H), lambda k: (0, 0)),
            pl.BlockSpec((H, 4 * H), lambda k: (0, 0)),
            pl.BlockSpec((1, 4 * H), lambda k: (0, 0)),
        ],
        out_specs=[
            pl.BlockSpec((B, _CHUNK, H), lambda k: (0, k, 0)),
            pl.BlockSpec((B, _CHUNK, H), lambda k: (0, nk - 1 - k, 0)),
        ],
        out_shape=[
            jax.ShapeDtypeStruct((B, T, H), jnp.float32),
            jax.ShapeDtypeStruct((B, T, H), jnp.float32),
        ],
        scratch_shapes=[pltpu.VMEM((B, H), jnp.float32)] * 4,
        compiler_params=pltpu.CompilerParams(
            dimension_semantics=("arbitrary",),
        ),
    )(sentences, sentences, wih_f, whh_f, bias_f, wih_b, whh_b, bias_b)

    lstm_out, vals, idx = pl.pallas_call(
        _align_topk_kernel,
        grid=(B // _GB,),
        in_specs=[
            pl.BlockSpec((_GB, T, H), lambda b: (b, 0, 0)),
            pl.BlockSpec((_GB, T, H), lambda b: (b, 0, 0)),
        ],
        out_specs=[
            pl.BlockSpec((_GB, T, 2 * H), lambda b: (b, 0, 0)),
            pl.BlockSpec((_GB, T, KPAD), lambda b: (b, 0, 0)),
            pl.BlockSpec((_GB, T, KPAD), lambda b: (b, 0, 0)),
        ],
        out_shape=[
            jax.ShapeDtypeStruct((B, T, 2 * H), jnp.float32),
            jax.ShapeDtypeStruct((B, T, KPAD), jnp.float32),
            jax.ShapeDtypeStruct((B, T, KPAD), jnp.int32),
        ],
        compiler_params=pltpu.CompilerParams(
            dimension_semantics=("parallel",),
        ),
    )(out_f, out_b)

    adj = idx[:, :, :3].reshape(B, T * 3)
    row1 = jnp.broadcast_to(
        jnp.repeat(jnp.arange(T, dtype=jnp.int32), 3)[None, :], (B, T * 3))
    coo = jnp.stack([adj, row1], axis=1)
    return (coo, vals[:, :, :3], lstm_out)


# speedup vs baseline: 1.7368x; 1.0369x over previous
import math

import jax
import jax.numpy as jnp
from jax import lax
from jax.experimental import pallas as pl
from jax.experimental.pallas import tpu as pltpu

T, B, I, H = 128, 128, 512, 512
KPAD = 8


_CHUNK = 8


def _bilstm_kernel(xf_ref, xb_ref, wih_f_ref, whh_f_ref, bf_ref,
                   wih_b_ref, whh_b_ref, bb_ref,
                   outf_ref, outb_ref, hf, cf, hb, cb):
    k = pl.program_id(0)

    @pl.when(k == 0)
    def _init():
        hf[...] = jnp.zeros_like(hf)
        cf[...] = jnp.zeros_like(cf)
        hb[...] = jnp.zeros_like(hb)
        cb[...] = jnp.zeros_like(cb)

    xf = xf_ref[...].reshape(_CHUNK * B, I)
    xb = xb_ref[...].reshape(_CHUNK * B, I)

    def _cell(x, wih_ref, whh_ref, b_ref, h, c):
        g = (jnp.dot(x, wih_ref[...], preferred_element_type=jnp.float32)
             + jnp.dot(h[...], whh_ref[...], preferred_element_type=jnp.float32)
             + b_ref[...])
        ig = jax.nn.sigmoid(g[:, 0:H])
        fg = jax.nn.sigmoid(g[:, H:2 * H])
        gg = jnp.tanh(g[:, 2 * H:3 * H])
        og = jax.nn.sigmoid(g[:, 3 * H:4 * H])
        c_new = fg * c[...] + ig * gg
        h_new = og * jnp.tanh(c_new)
        c[...] = c_new
        h[...] = h_new
        return h_new

    hs_f, hs_b = [], []
    for i in range(_CHUNK):
        hs_f.append(_cell(xf[i * B:(i + 1) * B],
                          wih_f_ref, whh_f_ref, bf_ref, hf, cf))
        hs_b.append(_cell(xb[(_CHUNK - 1 - i) * B:(_CHUNK - i) * B],
                          wih_b_ref, whh_b_ref, bb_ref, hb, cb))

    outf_ref[...] = jnp.stack(hs_f, axis=1)
    outb_ref[...] = jnp.stack(hs_b[::-1], axis=1)


_GB = 8


def _align_topk_kernel(f_ref, b_ref, lstm_ref, vals_ref, idx_ref):
    dn = (((1,), (1,)), ((), ()))
    iota = lax.broadcasted_iota(jnp.int32, (T, T), 1)
    neg = jnp.float32(-3e38)
    for g in range(_GB):
        lf = f_ref[g]
        lb = b_ref[g]
        lstm_ref[g, :, 0:H] = lf
        lstm_ref[g, :, H:2 * H] = lb
        a = (lax.dot_general(lf, lf, dn, preferred_element_type=jnp.float32)
             + lax.dot_general(lb, lb, dn, preferred_element_type=jnp.float32))
        a = a * (1.0 / math.sqrt(2 * H))
        vals_ref[g] = jnp.zeros((T, KPAD), jnp.float32)
        idx_ref[g] = jnp.zeros((T, KPAD), jnp.int32)
        for j in range(3):
            m = jnp.max(a, axis=1, keepdims=True)
            sel = jnp.where(a == m, iota, T)
            ix = jnp.min(sel, axis=1, keepdims=True)
            vals_ref[g, :, j:j + 1] = m
            idx_ref[g, :, j:j + 1] = ix
            if j < 2:
                a = jnp.where(iota == ix, neg, a)


def kernel(sentences, W_ih_f, W_hh_f, b_ih_f, b_hh_f,
           W_ih_b, W_hh_b, b_ih_b, b_hh_b):
    wih_f = W_ih_f.T
    whh_f = W_hh_f.T
    wih_b = W_ih_b.T
    whh_b = W_hh_b.T
    bias_f = (b_ih_f + b_hh_f).reshape(1, 4 * H)
    bias_b = (b_ih_b + b_hh_b).reshape(1, 4 * H)

    nk = T // _CHUNK
    out_f, out_b = pl.pallas_call(
        _bilstm_kernel,
        grid=(nk,),
        in_specs=[
            pl.BlockSpec((_CHUNK, B, I), lambda k: (k, 0, 0)),
            pl.BlockSpec((_CHUNK, B, I), lambda k: (nk - 1 - k, 0, 0)),
            pl.BlockSpec((I, 4 * H), lambda k: (0, 0)),
            pl.BlockSpec((H, 4 * H), lambda k: (0, 0)),
            pl.BlockSpec((1, 4 * H), lambda k: (0, 0)),
            pl.BlockSpec((I, 4 * H), lambda k: (0, 0)),
            pl.BlockSpec((H, 4 * H), lambda k: (0, 0)),
            pl.BlockSpec((1, 4 * H), lambda k: (0, 0)),
        ],
        out_specs=[
            pl.BlockSpec((B, _CHUNK, H), lambda k: (0, k, 0)),
            pl.BlockSpec((B, _CHUNK, H), lambda k: (0, nk - 1 - k, 0)),
        ],
        out_shape=[
            jax.ShapeDtypeStruct((B, T, H), jnp.float32),
            jax.ShapeDtypeStruct((B, T, H), jnp.float32),
        ],
        scratch_shapes=[pltpu.VMEM((B, H), jnp.float32)] * 4,
        compiler_params=pltpu.CompilerParams(
            dimension_semantics=("arbitrary",),
        ),
    )(sentences, sentences, wih_f, whh_f, bias_f, wih_b, whh_b, bias_b)

    lstm_out, vals, idx = pl.pallas_call(
        _align_topk_kernel,
        grid=(B // _GB,),
        in_specs=[
            pl.BlockSpec((_GB, T, H), lambda b: (b, 0, 0)),
            pl.BlockSpec((_GB, T, H), lambda b: (b, 0, 0)),
        ],
        out_specs=[
            pl.BlockSpec((_GB, T, 2 * H), lambda b: (b, 0, 0)),
            pl.BlockSpec((_GB, T, KPAD), lambda b: (b, 0, 0)),
            pl.BlockSpec((_GB, T, KPAD), lambda b: (b, 0, 0)),
        ],
        out_shape=[
            jax.ShapeDtypeStruct((B, T, 2 * H), jnp.float32),
            jax.ShapeDtypeStruct((B, T, KPAD), jnp.float32),
            jax.ShapeDtypeStruct((B, T, KPAD), jnp.int32),
        ],
        compiler_params=pltpu.CompilerParams(
            dimension_semantics=("parallel",),
        ),
    )(out_f, out_b)

    adj = idx[:, :, :3].reshape(B, T * 3)
    row1 = jnp.broadcast_to(
        jnp.repeat(jnp.arange(T, dtype=jnp.int32), 3)[None, :], (B, T * 3))
    coo = jnp.stack([adj, row1], axis=1)
    return (coo, vals[:, :, :3], lstm_out)
